# Initial kernel scaffold; baseline (speedup 1.0000x reference)
import jax
import jax.numpy as jnp
from jax.experimental import pallas as pl


def kernel(x, W_hm, W_wh, W_reg):
    raise NotImplementedError("write your pallas kernel here")


# speedup vs baseline: 4.1590x; 4.1590x over previous
import jax
import jax.numpy as jnp
from jax.experimental import pallas as pl
from jax.experimental.pallas import tpu as pltpu

B, C_IN, HF, WF = 8, 64, 128, 128
NUM_CLASSES = 80
K = 100
HW = HF * WF
NROWS = NUM_CLASSES * HF
NEG = -1e30
BIGI = 2**30


def _det_kernel(x_ref, whm_ref, wrw_ref, out_ref, hm_scr, rw_scr,
                cand_v, cand_g, res_scr):
    xb = x_ref[0]

    z = jnp.dot(whm_ref[...], xb, preferred_element_type=jnp.float32)
    rw = jnp.dot(wrw_ref[...], xb, preferred_element_type=jnp.float32)
    rw_scr[...] = rw.reshape(4 * HF, WF)

    z3 = z.reshape(NUM_CLASSES, HF, WF)
    negw = jnp.full((NUM_CLASSES, HF, 1), NEG, jnp.float32)
    zl = jnp.concatenate([z3[:, :, 1:], negw], axis=2)
    zr = jnp.concatenate([negw, z3[:, :, :-1]], axis=2)
    mw = jnp.maximum(jnp.maximum(zl, zr), z3)
    negh = jnp.full((NUM_CLASSES, 1, WF), NEG, jnp.float32)
    mu = jnp.concatenate([mw[:, 1:, :], negh], axis=1)
    md = jnp.concatenate([negh, mw[:, :-1, :]], axis=1)
    hmax = jnp.maximum(jnp.maximum(mu, md), mw)
    znms = jnp.where(hmax == z3, z3, NEG)

    hm_scr[...] = znms.reshape(NROWS, WF)
    rowmax = jnp.max(znms, axis=2).reshape(NUM_CLASSES, HF)

    ridx = (jax.lax.broadcasted_iota(jnp.int32, (NUM_CLASSES, HF), 0) * HF
            + jax.lax.broadcasted_iota(jnp.int32, (NUM_CLASSES, HF), 1))
    col = jax.lax.broadcasted_iota(jnp.int32, (1, WF), 1)

    def body_a(i, vals):
        m = jnp.max(vals)
        r = jnp.min(jnp.where(vals == m, ridx, BIGI))
        cand_v[pl.ds(i, 1), :] = hm_scr[pl.ds(r, 1), :]
        cand_g[pl.ds(i, 1), :] = r * WF + col
        return jnp.where(ridx == r, -jnp.inf, vals)

    jax.lax.fori_loop(0, K, body_a, rowmax)

    gidx = cand_g[...]
    lane = jax.lax.broadcasted_iota(jnp.int32, (1, WF), 1)

    def body_b(j, vals):
        m = jnp.max(vals)
        g = jnp.min(jnp.where(vals == m, gidx, BIGI))
        c = g // HW
        sp = g - c * HW
        yy = sp // WF
        xx = sp - yy * WF
        onehot = (lane == xx).astype(jnp.float32)
        regx = jnp.sum(rw_scr[pl.ds(yy, 1), :] * onehot)
        regy = jnp.sum(rw_scr[pl.ds(HF + yy, 1), :] * onehot)
        ww = jnp.sum(rw_scr[pl.ds(2 * HF + yy, 1), :] * onehot)
        hh = jnp.sum(rw_scr[pl.ds(3 * HF + yy, 1), :] * onehot)
        score = jax.nn.sigmoid(m)
        xs = xx.astype(jnp.float32) + regx
        ys = yy.astype(jnp.float32) + regy
        row =((lane == 0) * (xs - ww * 0.5) + (lane == 1) * (ys - hh * 0.5)
               + (lane == 2) * (xs + ww * 0.5) + (lane == 3) * (ys + hh * 0.5)
               + (lane == 4) * score + (lane == 5) * c.astype(jnp.float32))
        res_scr[pl.ds(j, 1), :] = row.astype(jnp.float32)
        return jnp.where(gidx == g, -jnp.inf, vals)

    jax.lax.fori_loop(0, K, body_b, cand_v[...])
    out_ref[0, :, :] = res_scr[:, :6]


@jax.jit
def kernel(x, W_hm, W_wh, W_reg):
    xf = x.reshape(B, C_IN, HW)
    wrw = jnp.concatenate([W_reg, W_wh], axis=0)
    dets = pl.pallas_call(
        _det_kernel,
        grid=(B,),
        in_specs=[
            pl.BlockSpec((1, C_IN, HW), lambda b: (b, 0, 0)),
            pl.BlockSpec((NUM_CLASSES, C_IN), lambda b: (0, 0)),
            pl.BlockSpec((4, C_IN), lambda b: (0, 0)),
        ],
        out_specs=pl.BlockSpec((1, K, 6), lambda b: (b, 0, 0)),
        out_shape=jax.ShapeDtypeStruct((B, K, 6), jnp.float32),
        scratch_shapes=[
            pltpu.VMEM((NROWS, WF), jnp.float32),
            pltpu.VMEM((4 * HF, WF), jnp.float32),
            pltpu.VMEM((K, WF), jnp.float32),
            pltpu.VMEM((K, WF), jnp.int32),
            pltpu.VMEM((K, WF), jnp.float32),
        ],
        compiler_params=pltpu.CompilerParams(
            dimension_semantics=("arbitrary",),
        ),
    )(xf, W_hm, wrw)
    return dets


# speedup vs baseline: 4.1591x; 1.0000x over previous
import jax
import jax.numpy as jnp
from jax.experimental import pallas as pl
from jax.experimental.pallas import tpu as pltpu

B, C_IN, HF, WF = 8, 64, 128, 128
NUM_CLASSES = 80
K = 100
HW = HF * WF
NROWS = NUM_CLASSES * HF
NEG = -1e30
BIGI = 2**30


def _det_kernel(x_ref, whm_ref, wrw_ref, out_ref, hm_scr, rw_scr,
                cand_v, cand_g, res_scr):
    xb = x_ref[0]

    z = jnp.dot(whm_ref[...], xb, preferred_element_type=jnp.float32)
    rw = jnp.dot(wrw_ref[...], xb, preferred_element_type=jnp.float32)
    rw_scr[...] = rw.reshape(4 * HF, WF)

    z3 = z.reshape(NUM_CLASSES, HF, WF)
    negw = jnp.full((NUM_CLASSES, HF, 1), NEG, jnp.float32)
    zl = jnp.concatenate([z3[:, :, 1:], negw], axis=2)
    zr = jnp.concatenate([negw, z3[:, :, :-1]], axis=2)
    mw = jnp.maximum(jnp.maximum(zl, zr), z3)
    negh = jnp.full((NUM_CLASSES, 1, WF), NEG, jnp.float32)
    mu = jnp.concatenate([mw[:, 1:, :], negh], axis=1)
    md = jnp.concatenate([negh, mw[:, :-1, :]], axis=1)
    hmax = jnp.maximum(jnp.maximum(mu, md), mw)
    znms = jnp.where(hmax == z3, z3, NEG)

    hm_scr[...] = znms.reshape(NROWS, WF)
    rowmax = jnp.max(znms, axis=2).reshape(NUM_CLASSES, HF)

    ridx = (jax.lax.broadcasted_iota(jnp.int32, (NUM_CLASSES, HF), 0) * HF
            + jax.lax.broadcasted_iota(jnp.int32, (NUM_CLASSES, HF), 1))
    col = jax.lax.broadcasted_iota(jnp.int32, (1, WF), 1)

    def body_a(i, vals):
        m = jnp.max(vals)
        r = jnp.min(jnp.where(vals == m, ridx, BIGI))
        cand_v[pl.ds(i, 1), :] = hm_scr[pl.ds(r, 1), :]
        cand_g[pl.ds(i, 1), :] = r * WF + col
        return jnp.where(ridx == r, -jnp.inf, vals)

    jax.lax.fori_loop(0, K, body_a, rowmax)

    gidx = cand_g[...]
    lane = jax.lax.broadcasted_iota(jnp.int32, (1, WF), 1)

    def body_b(j, vals):
        m = jnp.max(vals)
        g = jnp.min(jnp.where(vals == m, gidx, BIGI))
        c = g // HW
        sp = g - c * HW
        yy = sp // WF
        xx = sp - yy * WF
        onehot = (lane == xx).astype(jnp.float32)
        regx = jnp.sum(rw_scr[pl.ds(yy, 1), :] * onehot)
        regy = jnp.sum(rw_scr[pl.ds(HF + yy, 1), :] * onehot)
        ww = jnp.sum(rw_scr[pl.ds(2 * HF + yy, 1), :] * onehot)
        hh = jnp.sum(rw_scr[pl.ds(3 * HF + yy, 1), :] * onehot)
        score = jax.nn.sigmoid(m)
        xs = xx.astype(jnp.float32) + regx
        ys = yy.astype(jnp.float32) + regy
        row =((lane == 0) * (xs - ww * 0.5) + (lane == 1) * (ys - hh * 0.5)
               + (lane == 2) * (xs + ww * 0.5) + (lane == 3) * (ys + hh * 0.5)
               + (lane == 4) * score + (lane == 5) * c.astype(jnp.float32))
        res_scr[pl.ds(j, 1), :] = row.astype(jnp.float32)
        return jnp.where(gidx == g, -jnp.inf, vals)

    jax.lax.fori_loop(0, K, body_b, cand_v[...])
    out_ref[0, :, :] = res_scr[:, :6]


@jax.jit
def kernel(x, W_hm, W_wh, W_reg):
    xf = x.reshape(B, C_IN, HW)
    wrw = jnp.concatenate([W_reg, W_wh], axis=0)
    dets = pl.pallas_call(
        _det_kernel,
        grid=(B,),
        in_specs=[
            pl.BlockSpec((1, C_IN, HW), lambda b: (b, 0, 0)),
            pl.BlockSpec((NUM_CLASSES, C_IN), lambda b: (0, 0)),
            pl.BlockSpec((4, C_IN), lambda b: (0, 0)),
        ],
        out_specs=pl.BlockSpec((1, K, 6), lambda b: (b, 0, 0)),
        out_shape=jax.ShapeDtypeStruct((B, K, 6), jnp.float32),
        scratch_shapes=[
            pltpu.VMEM((NROWS, WF), jnp.float32),
            pltpu.VMEM((4 * HF, WF), jnp.float32),
            pltpu.VMEM((K, WF), jnp.float32),
            pltpu.VMEM((K, WF), jnp.int32),
            pltpu.VMEM((K, WF), jnp.float32),
        ],
        compiler_params=pltpu.CompilerParams(
            dimension_semantics=("parallel",),
        ),
    )(xf, W_hm, wrw)
    return dets
